# Initial kernel scaffold; baseline (speedup 1.0000x reference)
import jax
import jax.numpy as jnp
from jax.experimental import pallas as pl


def kernel(x, edge_index, mask_road, mask_rail, W_road, b_road, W_rail, b_rail):
    raise NotImplementedError("write your pallas kernel here")


# speedup vs baseline: 2.4144x; 2.4144x over previous
import functools

import jax
import jax.numpy as jnp
from jax import lax
from jax.experimental import pallas as pl
from jax.experimental.pallas import tpu as pltpu
from jax.experimental.pallas import tpu_sc as plsc

N = 10000
NPAD = 10240
D = 128
E = 320000
EPAD = 327680
TPT = EPAD // 16
NCHUNK = TPT // 128
RPT = NPAD // 16
DUMMY = N

_MESH = plsc.VectorSubcoreMesh(core_axis_name="c", subcore_axis_name="s")
_SC_PARAMS = pltpu.CompilerParams(needs_layout_passes=False)


def _deg_body(dst_hbm, maski_hbm, deg_out, dst_v, m_v, degp_v, red_v, sum_v,
              degsh):
    c = lax.axis_index("c")
    s = lax.axis_index("s")
    zeros16 = jnp.zeros((16,), jnp.float32)

    def _zero(i, _):
        degp_v[pl.ds(i * 16, 16)] = zeros16
        return _

    lax.fori_loop(0, RPT, _zero, None)

    ebase = s * TPT
    mbase = c * EPAD + ebase
    for k in range(NCHUNK // 16):
        off = k * 2048
        pltpu.sync_copy(dst_hbm.at[pl.ds(ebase + off, 2048)], dst_v)
        pltpu.sync_copy(maski_hbm.at[pl.ds(mbase + off, 2048)], m_v)

        def _scat(i, _):
            d16 = dst_v[pl.ds(i * 16, 16)]
            m16 = m_v[pl.ds(i * 16, 16)].astype(jnp.float32)
            plsc.addupdate_scatter(degp_v, [d16], m16)
            return _

        lax.fori_loop(0, 128, _scat, None)

    pltpu.sync_copy(degp_v, degsh.at[s])
    plsc.subcore_barrier()
    pltpu.sync_copy(degsh.at[:, pl.ds(s * RPT, RPT)], red_v)

    def _red(j, _):
        sl = pl.ds(j * 16, 16)
        acc = red_v[0, sl]
        for r in range(1, 16):
            acc = acc + red_v[r, sl]
        sum_v[sl] = acc
        return _

    lax.fori_loop(0, RPT // 16, _red, None)
    pltpu.sync_copy(sum_v, deg_out.at[c, pl.ds(s * RPT, RPT)])


@functools.partial(
    pl.kernel,
    out_type=jax.ShapeDtypeStruct((2, NPAD), jnp.float32),
    mesh=_MESH,
    scratch_types=[
        pltpu.VMEM((2048,), jnp.int32),
        pltpu.VMEM((2048,), jnp.int32),
        pltpu.VMEM((NPAD,), jnp.float32),
        pltpu.VMEM((16, RPT), jnp.float32),
        pltpu.VMEM((RPT,), jnp.float32),
        pltpu.VMEM_SHARED((16, NPAD), jnp.float32),
    ],
    compiler_params=_SC_PARAMS,
)
def _deg_kernel(dst_hbm, maski_hbm, deg_out, *rest):
    _deg_body(dst_hbm, maski_hbm, deg_out, *rest)


def _prep_body(x_ref, deg_ref, y0_ref, inv_ref):
    d = jnp.maximum(deg_ref[...], 1.0)
    inv_ref[...] = 1.0 / d
    norm = jnp.reshape(lax.rsqrt(d), (2, 1024))
    xb = x_ref[...]
    y0_ref[0] = xb * norm[0][:, None]
    y0_ref[1] = xb * norm[1][:, None]


def _tc_prep(x_pad, deg3):
    return pl.pallas_call(
        _prep_body,
        grid=(NPAD // 1024,),
        in_specs=[
            pl.BlockSpec((1024, D), lambda i: (i, 0)),
            pl.BlockSpec((2, 8, 128), lambda i: (0, i, 0)),
        ],
        out_specs=[
            pl.BlockSpec((2, 1024, D), lambda i: (0, i, 0)),
            pl.BlockSpec((2, 8, 128), lambda i: (0, i, 0)),
        ],
        out_shape=[
            jax.ShapeDtypeStruct((2, NPAD, D), jnp.float32),
            jax.ShapeDtypeStruct((2, NPAD // 128, 128), jnp.float32),
        ],
    )(x_pad, deg3)


def _hop(y_ref, src_hbm, dst_hbm, maski_hbm, acc_sh, src_v, dst_v, msk_v,
         gsrc_v, dstp_v, rows_v, sem, c, s):
    ebase = s * TPT
    mbase = c * EPAD + ebase
    goff = c * NPAD

    def _chunk(t, _):
        off = t * 128
        pltpu.sync_copy(src_hbm.at[pl.ds(ebase + off, 128)], src_v)
        pltpu.sync_copy(dst_hbm.at[pl.ds(ebase + off, 128)], dst_v)
        pltpu.sync_copy(maski_hbm.at[pl.ds(mbase + off, 128)], msk_v)
        for j in range(8):
            sl = pl.ds(j * 16, 16)
            d16 = dst_v[sl]
            m16 = msk_v[sl]
            gsrc_v[sl] = src_v[sl] + goff
            dstp_v[sl] = jnp.where(m16 != 0, d16, DUMMY)
        pltpu.async_copy(y_ref.at[gsrc_v], rows_v, sem).wait()
        pltpu.sync_copy(rows_v, acc_sh.at[dstp_v], add=True)
        return _

    lax.fori_loop(0, NCHUNK, _chunk, None)


def _hops_body(y0_hbm, src_hbm, dst_hbm, maski_hbm, inv_hbm,
               s1_out, s2_out, y1_out,
               src_v, dst_v, msk_v, gsrc_v, dstp_v, rows_v, zeros_v, inv_v,
               sem, acc_sh):
    c = lax.axis_index("c")
    s = lax.axis_index("s")
    zeros16 = jnp.zeros((16,), jnp.float32)

    def _zrow(i, _):
        for j in range(8):
            zeros_v[i, pl.ds(j * 16, 16)] = zeros16
        return _

    lax.fori_loop(0, 128, _zrow, None)
    for kk in range(RPT // 128):
        pltpu.sync_copy(zeros_v, acc_sh.at[pl.ds(s * RPT + kk * 128, 128)])
    plsc.subcore_barrier()

    hop_args = (src_hbm, dst_hbm, maski_hbm, acc_sh, src_v, dst_v, msk_v,
                gsrc_v, dstp_v, rows_v, sem, c, s)
    _hop(y0_hbm, *hop_args)
    plsc.subcore_barrier()

    rbase = c * NPAD + s * RPT
    pltpu.sync_copy(inv_hbm.at[pl.ds(rbase, RPT)], inv_v)
    for kk in range(RPT // 128):
        lrow = s * RPT + kk * 128
        hrow = rbase + kk * 128
        pltpu.sync_copy(acc_sh.at[pl.ds(lrow, 128)], rows_v)
        pltpu.sync_copy(rows_v, s1_out.at[pl.ds(hrow, 128)])

        def _scale(g, _, kk=kk):
            iv16 = inv_v[pl.ds(kk * 128 + g * 16, 16)]
            for l in range(16):
                sc = iv16[l]
                r = g * 16 + l
                for j in range(8):
                    sl = pl.ds(j * 16, 16)
                    rows_v[r, sl] = rows_v[r, sl] * sc
            return _

        lax.fori_loop(0, 8, _scale, None)
        pltpu.sync_copy(rows_v, y1_out.at[pl.ds(hrow, 128)])
        pltpu.sync_copy(zeros_v, acc_sh.at[pl.ds(lrow, 128)])
    plsc.subcore_barrier()

    _hop(y1_out, *hop_args)
    plsc.subcore_barrier()

    for kk in range(RPT // 128):
        pltpu.sync_copy(acc_sh.at[pl.ds(s * RPT + kk * 128, 128)], rows_v)
        pltpu.sync_copy(rows_v, s2_out.at[pl.ds(rbase + kk * 128, 128)])


@functools.partial(
    pl.kernel,
    out_type=[
        jax.ShapeDtypeStruct((2 * NPAD, D), jnp.float32),
        jax.ShapeDtypeStruct((2 * NPAD, D), jnp.float32),
        jax.ShapeDtypeStruct((2 * NPAD, D), jnp.float32),
    ],
    mesh=_MESH,
    scratch_types=[
        pltpu.VMEM((128,), jnp.int32),
        pltpu.VMEM((128,), jnp.int32),
        pltpu.VMEM((128,), jnp.int32),
        pltpu.VMEM((128,), jnp.int32),
        pltpu.VMEM((128,), jnp.int32),
        pltpu.VMEM((128, D), jnp.float32),
        pltpu.VMEM((128, D), jnp.float32),
        pltpu.VMEM((RPT,), jnp.float32),
        pltpu.SemaphoreType.DMA,
        pltpu.VMEM_SHARED((NPAD, D), jnp.float32),
    ],
    compiler_params=_SC_PARAMS,
)
def _hops_kernel(*args):
    _hops_body(*args)


def _final_body(x_ref, s1_ref, s2_ref, deg_ref, wx_ref, w1r_ref, w2r_ref,
                w1l_ref, w2l_ref, b_ref, out_ref):
    n = jnp.reshape(lax.rsqrt(jnp.maximum(deg_ref[...], 1.0)), (2, 1024))
    xb = x_ref[...]
    acc = jnp.dot(xb, wx_ref[...], preferred_element_type=jnp.float32)
    pr = (jnp.dot(s1_ref[0], w1r_ref[...], preferred_element_type=jnp.float32)
          + jnp.dot(s2_ref[0], w2r_ref[...], preferred_element_type=jnp.float32))
    pll = (jnp.dot(s1_ref[1], w1l_ref[...], preferred_element_type=jnp.float32)
           + jnp.dot(s2_ref[1], w2l_ref[...], preferred_element_type=jnp.float32))
    scaled = jnp.concatenate(
        [pr * n[0][:, None], pll * n[1][:, None]], axis=1)
    out_ref[...] = jnp.maximum(acc + scaled + b_ref[...], 0.0)


def _tc_final(x_pad, s1, s2, deg3, wx, w1r, w2r, w1l, w2l, b_all):
    wspec = pl.BlockSpec((D, 64), lambda i: (0, 0))
    return pl.pallas_call(
        _final_body,
        grid=(NPAD // 1024,),
        in_specs=[
            pl.BlockSpec((1024, D), lambda i: (i, 0)),
            pl.BlockSpec((2, 1024, D), lambda i: (0, i, 0)),
            pl.BlockSpec((2, 1024, D), lambda i: (0, i, 0)),
            pl.BlockSpec((2, 8, 128), lambda i: (0, i, 0)),
            pl.BlockSpec((D, D), lambda i: (0, 0)),
            wspec, wspec, wspec, wspec,
            pl.BlockSpec((1, D), lambda i: (0, 0)),
        ],
        out_specs=pl.BlockSpec((1024, D), lambda i: (i, 0)),
        out_shape=jax.ShapeDtypeStruct((NPAD, D), jnp.float32),
    )(x_pad, s1, s2, deg3, wx, w1r, w2r, w1l, w2l, b_all)


def kernel(x, edge_index, mask_road, mask_rail, W_road, b_road, W_rail,
           b_rail):
    src_pad = jnp.pad(edge_index[0], (0, EPAD - E))
    dst_pad = jnp.pad(edge_index[1], (0, EPAD - E))
    maski = jnp.pad(
        jnp.stack([mask_road, mask_rail]).astype(jnp.int32),
        ((0, 0), (0, EPAD - E))).reshape(-1)
    x_pad = jnp.pad(x, ((0, NPAD - N), (0, 0)))

    deg3 = _deg_kernel(dst_pad, maski).reshape(2, NPAD // 128, 128)
    y0, inv3 = _tc_prep(x_pad, deg3)
    s1f, s2f, _ = _hops_kernel(
        y0.reshape(2 * NPAD, D), src_pad, dst_pad, maski, inv3.reshape(-1))

    wx = jnp.concatenate([W_road[0:D], W_rail[0:D]], axis=1)
    b_all = jnp.concatenate([b_road, b_rail])[None, :]
    out = _tc_final(
        x_pad, s1f.reshape(2, NPAD, D), s2f.reshape(2, NPAD, D), deg3,
        wx, W_road[D:2 * D], W_road[2 * D:], W_rail[D:2 * D], W_rail[2 * D:],
        b_all)
    return out[:N]


# speedup vs baseline: 3.2117x; 1.3302x over previous
import functools

import jax
import jax.numpy as jnp
from jax import lax
from jax.experimental import pallas as pl
from jax.experimental.pallas import tpu as pltpu
from jax.experimental.pallas import tpu_sc as plsc

N = 10000
NPAD = 10240
D = 128
E = 320000
EPAD = 327680
TPT = EPAD // 16
NCHUNK = TPT // 128
RPT = NPAD // 16
DUMMY = N

_MESH = plsc.VectorSubcoreMesh(core_axis_name="c", subcore_axis_name="s")
_SC_PARAMS = pltpu.CompilerParams(needs_layout_passes=False)


def _deg_body(dst_hbm, maski_hbm, deg_out, dst_v, m_v, degp_v, red_v, sum_v,
              degsh):
    c = lax.axis_index("c")
    s = lax.axis_index("s")
    zeros16 = jnp.zeros((16,), jnp.float32)

    def _zero(i, _):
        degp_v[pl.ds(i * 16, 16)] = zeros16
        return _

    lax.fori_loop(0, RPT, _zero, None)

    ebase = s * TPT
    mbase = c * EPAD + ebase
    for k in range(NCHUNK // 16):
        off = k * 2048
        pltpu.sync_copy(dst_hbm.at[pl.ds(ebase + off, 2048)], dst_v)
        pltpu.sync_copy(maski_hbm.at[pl.ds(mbase + off, 2048)], m_v)

        def _scat(i, _):
            d16 = dst_v[pl.ds(i * 16, 16)]
            m16 = m_v[pl.ds(i * 16, 16)].astype(jnp.float32)
            plsc.addupdate_scatter(degp_v, [d16], m16)
            return _

        lax.fori_loop(0, 128, _scat, None)

    pltpu.sync_copy(degp_v, degsh.at[s])
    plsc.subcore_barrier()
    pltpu.sync_copy(degsh.at[:, pl.ds(s * RPT, RPT)], red_v)

    def _red(j, _):
        sl = pl.ds(j * 16, 16)
        acc = red_v[0, sl]
        for r in range(1, 16):
            acc = acc + red_v[r, sl]
        sum_v[sl] = acc
        return _

    lax.fori_loop(0, RPT // 16, _red, None)
    pltpu.sync_copy(sum_v, deg_out.at[c, pl.ds(s * RPT, RPT)])


@functools.partial(
    pl.kernel,
    out_type=jax.ShapeDtypeStruct((2, NPAD), jnp.float32),
    mesh=_MESH,
    scratch_types=[
        pltpu.VMEM((2048,), jnp.int32),
        pltpu.VMEM((2048,), jnp.int32),
        pltpu.VMEM((NPAD,), jnp.float32),
        pltpu.VMEM((16, RPT), jnp.float32),
        pltpu.VMEM((RPT,), jnp.float32),
        pltpu.VMEM_SHARED((16, NPAD), jnp.float32),
    ],
    compiler_params=_SC_PARAMS,
)
def _deg_kernel(dst_hbm, maski_hbm, deg_out, *rest):
    _deg_body(dst_hbm, maski_hbm, deg_out, *rest)


def _prep_body(x_ref, deg_ref, y0_ref, inv_ref):
    d = jnp.maximum(deg_ref[...], 1.0)
    inv_ref[...] = 1.0 / d
    norm = jnp.reshape(lax.rsqrt(d), (2, 1024))
    xb = x_ref[...]
    y0_ref[0] = xb * norm[0][:, None]
    y0_ref[1] = xb * norm[1][:, None]


def _tc_prep(x_pad, deg3):
    return pl.pallas_call(
        _prep_body,
        grid=(NPAD // 1024,),
        in_specs=[
            pl.BlockSpec((1024, D), lambda i: (i, 0)),
            pl.BlockSpec((2, 8, 128), lambda i: (0, i, 0)),
        ],
        out_specs=[
            pl.BlockSpec((2, 1024, D), lambda i: (0, i, 0)),
            pl.BlockSpec((2, 8, 128), lambda i: (0, i, 0)),
        ],
        out_shape=[
            jax.ShapeDtypeStruct((2, NPAD, D), jnp.float32),
            jax.ShapeDtypeStruct((2, NPAD // 128, 128), jnp.float32),
        ],
    )(x_pad, deg3)


CH = 80
NCH = TPT // CH
BLD = 1280


def _hop(y_ref, tbl, acc_sh, rows, gsrc_s, dstp_s, sgs, sss):
    def _unpack(t, b):
        for l in range(CH // 16):
            sl = pl.ds(l * 16, 16)
            p = tbl[pl.ds(t * CH + l * 16, 16)]
            gsrc_s[b][sl] = p & 0x7FFF
            dstp_s[b][sl] = lax.shift_right_logical(p, 15)

    def _gather(t, b):
        pltpu.async_copy(y_ref.at[gsrc_s[b]], rows[b], sgs[b])

    for b in range(2):
        _unpack(b, b)
        _gather(b, b)

    def _group(o, _):
        for b in range(2):
            pltpu.make_async_copy(y_ref.at[gsrc_s[b]], rows[b],
                                  sgs[b]).wait()
            pltpu.async_copy(rows[b], acc_sh.at[dstp_s[b]], sss[b],
                             add=True)
        for b in range(2):
            t = o * 2 + b

            @pl.when(t + 2 < NCH)
            def _():
                pltpu.make_async_copy(rows[b], acc_sh.at[dstp_s[b]],
                                      sss[b]).wait()
                _unpack(t + 2, b)
                _gather(t + 2, b)
        return _

    lax.fori_loop(0, NCH // 2, _group, None)
    for b in range(2):
        pltpu.make_async_copy(rows[b], acc_sh.at[dstp_s[b]], sss[b]).wait()


def _hops_body(y0_hbm, src_hbm, dst_hbm, maski_hbm, inv_hbm,
               s1_out, s2_out, y1_out,
               st_src, st_dst, st_msk, tbl,
               gsrc0, gsrc1, dstp0, dstp1,
               r0, r1, inv_v,
               sg0, sg1, ss0, ss1,
               acc_sh):
    c = lax.axis_index("c")
    s = lax.axis_index("s")
    ebase = s * TPT
    mbase = c * EPAD + ebase
    goff = c * NPAD
    rbase = c * NPAD + s * RPT
    rows = [r0, r1]
    gsrc_s = [gsrc0, gsrc1]
    dstp_s = [dstp0, dstp1]
    sgs = [sg0, sg1]
    sss = [ss0, ss1]
    zeros16 = jnp.zeros((16,), jnp.float32)

    for c2 in range(TPT // BLD):
        off = c2 * BLD
        pltpu.sync_copy(src_hbm.at[pl.ds(ebase + off, BLD)], st_src)
        pltpu.sync_copy(dst_hbm.at[pl.ds(ebase + off, BLD)], st_dst)
        pltpu.sync_copy(maski_hbm.at[pl.ds(mbase + off, BLD)], st_msk)

        def _bld(i, _, c2=c2):
            sl = pl.ds(i * 16, 16)
            g16 = st_src[sl] + goff
            d16 = jnp.where(st_msk[sl] != 0, st_dst[sl], DUMMY)
            tbl[pl.ds(c2 * BLD + i * 16, 16)] = g16 | (d16 << 15)
            return _

        lax.fori_loop(0, BLD // 16, _bld, None)

    def _zrow(i, _):
        for j in range(8):
            r0[i, pl.ds(j * 16, 16)] = zeros16
        return _

    lax.fori_loop(0, CH, _zrow, None)
    for kk in range(RPT // CH):
        pltpu.sync_copy(r0, acc_sh.at[pl.ds(s * RPT + kk * CH, CH)])
    plsc.subcore_barrier()

    _hop(y0_hbm, tbl, acc_sh, rows, gsrc_s, dstp_s, sgs, sss)
    plsc.subcore_barrier()

    pltpu.sync_copy(inv_hbm.at[pl.ds(rbase, RPT)], inv_v)
    lax.fori_loop(0, CH, _zrow, None)
    for kk in range(RPT // CH):
        lrow = s * RPT + kk * CH
        hrow = rbase + kk * CH
        pltpu.sync_copy(acc_sh.at[pl.ds(lrow, CH)], r1)
        pltpu.sync_copy(r1, s1_out.at[pl.ds(hrow, CH)])

        def _scale(g, _, kk=kk):
            iv16 = inv_v[pl.ds(kk * CH + g * 16, 16)]
            for l in range(16):
                sc = iv16[l]
                r = g * 16 + l
                for j in range(8):
                    sl = pl.ds(j * 16, 16)
                    r1[r, sl] = r1[r, sl] * sc
            return _

        lax.fori_loop(0, CH // 16, _scale, None)
        pltpu.sync_copy(r1, y1_out.at[pl.ds(hrow, CH)])
        pltpu.sync_copy(r0, acc_sh.at[pl.ds(lrow, CH)])
    plsc.subcore_barrier()

    _hop(y1_out, tbl, acc_sh, rows, gsrc_s, dstp_s, sgs, sss)
    plsc.subcore_barrier()

    for kk in range(RPT // CH):
        pltpu.sync_copy(acc_sh.at[pl.ds(s * RPT + kk * CH, CH)], r1)
        pltpu.sync_copy(r1, s2_out.at[pl.ds(rbase + kk * CH, CH)])


@functools.partial(
    pl.kernel,
    out_type=[
        jax.ShapeDtypeStruct((2 * NPAD, D), jnp.float32),
        jax.ShapeDtypeStruct((2 * NPAD, D), jnp.float32),
        jax.ShapeDtypeStruct((2 * NPAD, D), jnp.float32),
    ],
    mesh=_MESH,
    scratch_types=[
        pltpu.VMEM((BLD,), jnp.int32),
        pltpu.VMEM((BLD,), jnp.int32),
        pltpu.VMEM((BLD,), jnp.int32),
        pltpu.VMEM((TPT,), jnp.int32),
        pltpu.VMEM((CH,), jnp.int32),
        pltpu.VMEM((CH,), jnp.int32),
        pltpu.VMEM((CH,), jnp.int32),
        pltpu.VMEM((CH,), jnp.int32),
        pltpu.VMEM((CH, D), jnp.float32),
        pltpu.VMEM((CH, D), jnp.float32),
        pltpu.VMEM((RPT,), jnp.float32),
        pltpu.SemaphoreType.DMA,
        pltpu.SemaphoreType.DMA,
        pltpu.SemaphoreType.DMA,
        pltpu.SemaphoreType.DMA,
        pltpu.VMEM_SHARED((NPAD, D), jnp.float32),
    ],
    compiler_params=_SC_PARAMS,
)
def _hops_kernel(*args):
    _hops_body(*args)


def _final_body(x_ref, s1_ref, s2_ref, deg_ref, wx_ref, w1r_ref, w2r_ref,
                w1l_ref, w2l_ref, b_ref, out_ref):
    n = jnp.reshape(lax.rsqrt(jnp.maximum(deg_ref[...], 1.0)), (2, 1024))
    xb = x_ref[...]
    acc = jnp.dot(xb, wx_ref[...], preferred_element_type=jnp.float32)
    pr = (jnp.dot(s1_ref[0], w1r_ref[...], preferred_element_type=jnp.float32)
          + jnp.dot(s2_ref[0], w2r_ref[...], preferred_element_type=jnp.float32))
    pll = (jnp.dot(s1_ref[1], w1l_ref[...], preferred_element_type=jnp.float32)
           + jnp.dot(s2_ref[1], w2l_ref[...], preferred_element_type=jnp.float32))
    scaled = jnp.concatenate(
        [pr * n[0][:, None], pll * n[1][:, None]], axis=1)
    out_ref[...] = jnp.maximum(acc + scaled + b_ref[...], 0.0)


def _tc_final(x_pad, s1, s2, deg3, wx, w1r, w2r, w1l, w2l, b_all):
    wspec = pl.BlockSpec((D, 64), lambda i: (0, 0))
    return pl.pallas_call(
        _final_body,
        grid=(NPAD // 1024,),
        in_specs=[
            pl.BlockSpec((1024, D), lambda i: (i, 0)),
            pl.BlockSpec((2, 1024, D), lambda i: (0, i, 0)),
            pl.BlockSpec((2, 1024, D), lambda i: (0, i, 0)),
            pl.BlockSpec((2, 8, 128), lambda i: (0, i, 0)),
            pl.BlockSpec((D, D), lambda i: (0, 0)),
            wspec, wspec, wspec, wspec,
            pl.BlockSpec((1, D), lambda i: (0, 0)),
        ],
        out_specs=pl.BlockSpec((1024, D), lambda i: (i, 0)),
        out_shape=jax.ShapeDtypeStruct((NPAD, D), jnp.float32),
    )(x_pad, s1, s2, deg3, wx, w1r, w2r, w1l, w2l, b_all)


def kernel(x, edge_index, mask_road, mask_rail, W_road, b_road, W_rail,
           b_rail):
    src_pad = jnp.pad(edge_index[0], (0, EPAD - E))
    dst_pad = jnp.pad(edge_index[1], (0, EPAD - E))
    maski = jnp.pad(
        jnp.stack([mask_road, mask_rail]).astype(jnp.int32),
        ((0, 0), (0, EPAD - E))).reshape(-1)
    x_pad = jnp.pad(x, ((0, NPAD - N), (0, 0)))

    deg3 = _deg_kernel(dst_pad, maski).reshape(2, NPAD // 128, 128)
    y0, inv3 = _tc_prep(x_pad, deg3)
    s1f, s2f, _ = _hops_kernel(
        y0.reshape(2 * NPAD, D), src_pad, dst_pad, maski, inv3.reshape(-1))

    wx = jnp.concatenate([W_road[0:D], W_rail[0:D]], axis=1)
    b_all = jnp.concatenate([b_road, b_rail])[None, :]
    out = _tc_final(
        x_pad, s1f.reshape(2, NPAD, D), s2f.reshape(2, NPAD, D), deg3,
        wx, W_road[D:2 * D], W_road[2 * D:], W_rail[D:2 * D], W_rail[2 * D:],
        b_all)
    return out[:N]


# speedup vs baseline: 7.9046x; 2.4612x over previous
import functools

import jax
import jax.numpy as jnp
from jax import lax
from jax.experimental import pallas as pl
from jax.experimental.pallas import tpu as pltpu
from jax.experimental.pallas import tpu_sc as plsc

N = 10000
NPAD = 10240
D = 128
E = 320000
EPAD = 327680
TPT = EPAD // 16
NCHUNK = TPT // 128
RPT = NPAD // 16
DUMMY = N

_MESH = plsc.VectorSubcoreMesh(core_axis_name="c", subcore_axis_name="s")
_SC_PARAMS = pltpu.CompilerParams(needs_layout_passes=False)


def _deg_body(dst_hbm, maski_hbm, deg_out, dst_v, m_v, degp_v, red_v, sum_v,
              degsh):
    c = lax.axis_index("c")
    s = lax.axis_index("s")
    zeros16 = jnp.zeros((16,), jnp.float32)

    def _zero(i, _):
        degp_v[pl.ds(i * 16, 16)] = zeros16
        return _

    lax.fori_loop(0, RPT, _zero, None)

    ebase = s * TPT
    mbase = c * EPAD + ebase
    for k in range(NCHUNK // 16):
        off = k * 2048
        pltpu.sync_copy(dst_hbm.at[pl.ds(ebase + off, 2048)], dst_v)
        pltpu.sync_copy(maski_hbm.at[pl.ds(mbase + off, 2048)], m_v)

        def _scat(i, _):
            d16 = dst_v[pl.ds(i * 16, 16)]
            m16 = m_v[pl.ds(i * 16, 16)].astype(jnp.float32)
            plsc.addupdate_scatter(degp_v, [d16], m16)
            return _

        lax.fori_loop(0, 128, _scat, None)

    pltpu.sync_copy(degp_v, degsh.at[s])
    plsc.subcore_barrier()
    pltpu.sync_copy(degsh.at[:, pl.ds(s * RPT, RPT)], red_v)

    def _red(j, _):
        sl = pl.ds(j * 16, 16)
        acc = red_v[0, sl]
        for r in range(1, 16):
            acc = acc + red_v[r, sl]
        sum_v[sl] = acc
        return _

    lax.fori_loop(0, RPT // 16, _red, None)
    pltpu.sync_copy(sum_v, deg_out.at[c, pl.ds(s * RPT, RPT)])


@functools.partial(
    pl.kernel,
    out_type=jax.ShapeDtypeStruct((2, NPAD), jnp.float32),
    mesh=_MESH,
    scratch_types=[
        pltpu.VMEM((2048,), jnp.int32),
        pltpu.VMEM((2048,), jnp.int32),
        pltpu.VMEM((NPAD,), jnp.float32),
        pltpu.VMEM((16, RPT), jnp.float32),
        pltpu.VMEM((RPT,), jnp.float32),
        pltpu.VMEM_SHARED((16, NPAD), jnp.float32),
    ],
    compiler_params=_SC_PARAMS,
)
def _deg_kernel(dst_hbm, maski_hbm, deg_out, *rest):
    _deg_body(dst_hbm, maski_hbm, deg_out, *rest)


def _prep_body(x_ref, deg_ref, y0_ref, inv_ref):
    d = jnp.maximum(deg_ref[...], 1.0)
    inv_ref[...] = 1.0 / d
    norm = jnp.reshape(lax.rsqrt(d), (2, 1024))
    xb = x_ref[...]
    y0_ref[0] = xb * norm[0][:, None]
    y0_ref[1] = xb * norm[1][:, None]


def _tc_prep(x_pad, deg3):
    return pl.pallas_call(
        _prep_body,
        grid=(NPAD // 1024,),
        in_specs=[
            pl.BlockSpec((1024, D), lambda i: (i, 0)),
            pl.BlockSpec((2, 8, 128), lambda i: (0, i, 0)),
        ],
        out_specs=[
            pl.BlockSpec((2, 1024, D), lambda i: (0, i, 0)),
            pl.BlockSpec((2, 8, 128), lambda i: (0, i, 0)),
        ],
        out_shape=[
            jax.ShapeDtypeStruct((2, NPAD, D), jnp.float32),
            jax.ShapeDtypeStruct((2, NPAD // 128, 128), jnp.float32),
        ],
    )(x_pad, deg3)


CH = 80
NCH = TPT // CH
BLD = 1280


def _hop(y_ref, tbl, acc_sh, rows, gsrc_s, dstp_s, sgs, sss, nhalf):
    nch = nhalf * 2

    def _unpack(t, b):
        for l in range(CH // 16):
            sl = pl.ds(l * 16, 16)
            p = tbl[pl.ds(t * CH + l * 16, 16)]
            gsrc_s[b][sl] = p & 0x7FFF
            dstp_s[b][sl] = lax.shift_right_logical(p, 15)

    def _gather(t, b):
        pltpu.async_copy(y_ref.at[gsrc_s[b]], rows[b], sgs[b])

    for b in range(2):
        _unpack(b, b)
        _gather(b, b)

    def _group(o, _):
        for b in range(2):
            pltpu.make_async_copy(y_ref.at[gsrc_s[b]], rows[b],
                                  sgs[b]).wait()
            pltpu.async_copy(rows[b], acc_sh.at[dstp_s[b]], sss[b],
                             add=True)
        for b in range(2):
            t = o * 2 + b

            @pl.when(t + 2 < nch)
            def _():
                pltpu.make_async_copy(rows[b], acc_sh.at[dstp_s[b]],
                                      sss[b]).wait()
                _unpack(t + 2, b)
                _gather(t + 2, b)
        return _

    lax.fori_loop(0, nhalf, _group, None)
    for b in range(2):
        pltpu.make_async_copy(rows[b], acc_sh.at[dstp_s[b]], sss[b]).wait()


def _hops_body(y0_hbm, src_hbm, dst_hbm, maski_hbm, inv_hbm,
               s1_out, s2_out, y1_out,
               st_src, st_dst, st_msk, tbl,
               gsrc0, gsrc1, dstp0, dstp1,
               r0, r1, inv_v,
               sg0, sg1, ss0, ss1,
               acc_sh):
    c = lax.axis_index("c")
    s = lax.axis_index("s")
    ebase = s * TPT
    mbase = c * EPAD + ebase
    goff = c * NPAD
    rbase = c * NPAD + s * RPT
    rows = [r0, r1]
    gsrc_s = [gsrc0, gsrc1]
    dstp_s = [dstp0, dstp1]
    sgs = [sg0, sg1]
    sss = [ss0, ss1]
    zeros16 = jnp.zeros((16,), jnp.float32)

    cnt = jnp.int32(0)
    for c2 in range(TPT // BLD):
        off = c2 * BLD
        pltpu.sync_copy(src_hbm.at[pl.ds(ebase + off, BLD)], st_src)
        pltpu.sync_copy(dst_hbm.at[pl.ds(ebase + off, BLD)], st_dst)
        pltpu.sync_copy(maski_hbm.at[pl.ds(mbase + off, BLD)], st_msk)

        def _bld(i, cnt):
            sl = pl.ds(i * 16, 16)
            g16 = st_src[sl] + goff
            m = st_msk[sl] != 0
            packed = g16 | (st_dst[sl] << 15)
            plsc.store_compressed(tbl.at[pl.ds(cnt, 16)], packed, mask=m)
            return cnt + plsc.all_reduce_population_count(m)[0]

        cnt = lax.fori_loop(0, BLD // 16, _bld, cnt)

    dummy16 = jnp.full((16,), goff | (DUMMY << 15), jnp.int32)
    all16 = jnp.ones((16,), jnp.bool_)
    for k in range(2 * CH // 16):
        plsc.store_compressed(tbl.at[pl.ds(cnt + k * 16, 16)], dummy16,
                              mask=all16)
    nhalf = (cnt + 2 * CH) // (2 * CH)

    def _zrow(i, _):
        for j in range(8):
            r0[i, pl.ds(j * 16, 16)] = zeros16
        return _

    lax.fori_loop(0, CH, _zrow, None)
    for kk in range(RPT // CH):
        pltpu.sync_copy(r0, acc_sh.at[pl.ds(s * RPT + kk * CH, CH)])
    plsc.subcore_barrier()

    _hop(y0_hbm, tbl, acc_sh, rows, gsrc_s, dstp_s, sgs, sss, nhalf)
    plsc.subcore_barrier()

    pltpu.sync_copy(inv_hbm.at[pl.ds(rbase, RPT)], inv_v)
    lax.fori_loop(0, CH, _zrow, None)
    for kk in range(RPT // CH):
        lrow = s * RPT + kk * CH
        hrow = rbase + kk * CH
        pltpu.sync_copy(acc_sh.at[pl.ds(lrow, CH)], r1)
        pltpu.sync_copy(r1, s1_out.at[pl.ds(hrow, CH)])

        def _scale(g, _, kk=kk):
            iv16 = inv_v[pl.ds(kk * CH + g * 16, 16)]
            for l in range(16):
                sc = iv16[l]
                r = g * 16 + l
                for j in range(8):
                    sl = pl.ds(j * 16, 16)
                    r1[r, sl] = r1[r, sl] * sc
            return _

        lax.fori_loop(0, CH // 16, _scale, None)
        pltpu.sync_copy(r1, y1_out.at[pl.ds(hrow, CH)])
        pltpu.sync_copy(r0, acc_sh.at[pl.ds(lrow, CH)])
    plsc.subcore_barrier()

    _hop(y1_out, tbl, acc_sh, rows, gsrc_s, dstp_s, sgs, sss, nhalf)
    plsc.subcore_barrier()

    for kk in range(RPT // CH):
        pltpu.sync_copy(acc_sh.at[pl.ds(s * RPT + kk * CH, CH)], r1)
        pltpu.sync_copy(r1, s2_out.at[pl.ds(rbase + kk * CH, CH)])


@functools.partial(
    pl.kernel,
    out_type=[
        jax.ShapeDtypeStruct((2 * NPAD, D), jnp.float32),
        jax.ShapeDtypeStruct((2 * NPAD, D), jnp.float32),
        jax.ShapeDtypeStruct((2 * NPAD, D), jnp.float32),
    ],
    mesh=_MESH,
    scratch_types=[
        pltpu.VMEM((BLD,), jnp.int32),
        pltpu.VMEM((BLD,), jnp.int32),
        pltpu.VMEM((BLD,), jnp.int32),
        pltpu.VMEM((TPT + 4 * CH,), jnp.int32),
        pltpu.VMEM((CH,), jnp.int32),
        pltpu.VMEM((CH,), jnp.int32),
        pltpu.VMEM((CH,), jnp.int32),
        pltpu.VMEM((CH,), jnp.int32),
        pltpu.VMEM((CH, D), jnp.float32),
        pltpu.VMEM((CH, D), jnp.float32),
        pltpu.VMEM((RPT,), jnp.float32),
        pltpu.SemaphoreType.DMA,
        pltpu.SemaphoreType.DMA,
        pltpu.SemaphoreType.DMA,
        pltpu.SemaphoreType.DMA,
        pltpu.VMEM_SHARED((NPAD, D), jnp.float32),
    ],
    compiler_params=_SC_PARAMS,
)
def _hops_kernel(*args):
    _hops_body(*args)


def _final_body(x_ref, s1_ref, s2_ref, deg_ref, wx_ref, w1r_ref, w2r_ref,
                w1l_ref, w2l_ref, b_ref, out_ref):
    n = jnp.reshape(lax.rsqrt(jnp.maximum(deg_ref[...], 1.0)), (2, 1024))
    xb = x_ref[...]
    acc = jnp.dot(xb, wx_ref[...], preferred_element_type=jnp.float32)
    pr = (jnp.dot(s1_ref[0], w1r_ref[...], preferred_element_type=jnp.float32)
          + jnp.dot(s2_ref[0], w2r_ref[...], preferred_element_type=jnp.float32))
    pll = (jnp.dot(s1_ref[1], w1l_ref[...], preferred_element_type=jnp.float32)
           + jnp.dot(s2_ref[1], w2l_ref[...], preferred_element_type=jnp.float32))
    scaled = jnp.concatenate(
        [pr * n[0][:, None], pll * n[1][:, None]], axis=1)
    out_ref[...] = jnp.maximum(acc + scaled + b_ref[...], 0.0)


def _tc_final(x_pad, s1, s2, deg3, wx, w1r, w2r, w1l, w2l, b_all):
    wspec = pl.BlockSpec((D, 64), lambda i: (0, 0))
    return pl.pallas_call(
        _final_body,
        grid=(NPAD // 1024,),
        in_specs=[
            pl.BlockSpec((1024, D), lambda i: (i, 0)),
            pl.BlockSpec((2, 1024, D), lambda i: (0, i, 0)),
            pl.BlockSpec((2, 1024, D), lambda i: (0, i, 0)),
            pl.BlockSpec((2, 8, 128), lambda i: (0, i, 0)),
            pl.BlockSpec((D, D), lambda i: (0, 0)),
            wspec, wspec, wspec, wspec,
            pl.BlockSpec((1, D), lambda i: (0, 0)),
        ],
        out_specs=pl.BlockSpec((1024, D), lambda i: (i, 0)),
        out_shape=jax.ShapeDtypeStruct((NPAD, D), jnp.float32),
    )(x_pad, s1, s2, deg3, wx, w1r, w2r, w1l, w2l, b_all)


def kernel(x, edge_index, mask_road, mask_rail, W_road, b_road, W_rail,
           b_rail):
    src_pad = jnp.pad(edge_index[0], (0, EPAD - E))
    dst_pad = jnp.pad(edge_index[1], (0, EPAD - E))
    maski = jnp.pad(
        jnp.stack([mask_road, mask_rail]).astype(jnp.int32),
        ((0, 0), (0, EPAD - E))).reshape(-1)
    x_pad = jnp.pad(x, ((0, NPAD - N), (0, 0)))

    deg3 = _deg_kernel(dst_pad, maski).reshape(2, NPAD // 128, 128)
    y0, inv3 = _tc_prep(x_pad, deg3)
    s1f, s2f, _ = _hops_kernel(
        y0.reshape(2 * NPAD, D), src_pad, dst_pad, maski, inv3.reshape(-1))

    wx = jnp.concatenate([W_road[0:D], W_rail[0:D]], axis=1)
    b_all = jnp.concatenate([b_road, b_rail])[None, :]
    out = _tc_final(
        x_pad, s1f.reshape(2, NPAD, D), s2f.reshape(2, NPAD, D), deg3,
        wx, W_road[D:2 * D], W_road[2 * D:], W_rail[D:2 * D], W_rail[2 * D:],
        b_all)
    return out[:N]
